# Initial kernel scaffold; baseline (speedup 1.0000x reference)
import jax
import jax.numpy as jnp
from jax.experimental import pallas as pl


def kernel(x, edge_index, W1, b1, W2, b2, Wf1, bf1, Wf2, bf2):
    raise NotImplementedError("write your pallas kernel here")


# speedup vs baseline: 20.9701x; 20.9701x over previous
import functools

import jax
import jax.numpy as jnp
from jax import lax
from jax.experimental import pallas as pl
from jax.experimental.pallas import tpu as pltpu
from jax.experimental.pallas import tpu_sc as plsc

N_NODES = 10000
DIM = 128
FC = 64
NPAD = 10240
N_EDGES = 320000
NC, NS = 2, 16
NW = NC * NS
CHUNK = 64
CPW = 160
EPW = CPW * CHUNK
EPAD = EPW * NW
ROWS_PT = NPAD // NS
BT = 256

_mesh = plsc.VectorSubcoreMesh(core_axis_name="c", subcore_axis_name="s",
                               num_cores=NC, num_subcores=NS)


G = 32
NG = CPW // G


@functools.partial(
    pl.kernel,
    out_type=jax.ShapeDtypeStruct((NC, NPAD, DIM), jnp.float32),
    mesh=_mesh,
    scratch_types=[
        pltpu.VMEM_SHARED((NPAD, DIM), jnp.float32),
        pltpu.VMEM((2, G, 2, CHUNK), jnp.int32),
        pltpu.VMEM((CHUNK, DIM), jnp.float32),
        pltpu.VMEM((CHUNK, DIM), jnp.float32),
        pltpu.SemaphoreType.DMA,
        pltpu.SemaphoreType.DMA,
        pltpu.SemaphoreType.DMA,
        pltpu.SemaphoreType.DMA,
    ],
)
def _agg_kernel(sd_hbm, hs_hbm, zeros_hbm, out_hbm,
                acc, idx_v, buf0, buf1, sem0, sem1, semi0, semi1):
    c = lax.axis_index("c")
    s = lax.axis_index("s")
    w = s * NC + c
    pltpu.sync_copy(zeros_hbm.at[pl.ds(s * ROWS_PT, ROWS_PT)],
                    acc.at[pl.ds(s * ROWS_PT, ROWS_PT)])
    pltpu.async_copy(sd_hbm.at[w, pl.ds(0, G)], idx_v.at[0], semi0)
    pltpu.async_copy(sd_hbm.at[w, pl.ds(G, G)], idx_v.at[1], semi1)
    plsc.subcore_barrier()

    for g in range(NG):
        b = g % 2
        semi = semi0 if b == 0 else semi1
        pltpu.make_async_copy(sd_hbm.at[w, pl.ds(g * G, G)], idx_v.at[b],
                              semi).wait()
        pltpu.async_copy(hs_hbm.at[idx_v.at[b, 0, 0]], buf0, sem0)

        def step(jj, carry, b=b):
            j0 = jj * 2
            pltpu.async_copy(hs_hbm.at[idx_v.at[b, j0 + 1, 0]], buf1, sem1)
            pltpu.make_async_copy(hs_hbm.at[idx_v.at[b, j0, 0]], buf0,
                                  sem0).wait()
            pltpu.sync_copy(buf0, acc.at[idx_v.at[b, j0, 1]], add=True)

            @pl.when(jj + 1 < G // 2)
            def _():
                pltpu.async_copy(hs_hbm.at[idx_v.at[b, j0 + 2, 0]], buf0, sem0)

            pltpu.make_async_copy(hs_hbm.at[idx_v.at[b, j0 + 1, 0]], buf1,
                                  sem1).wait()
            pltpu.sync_copy(buf1, acc.at[idx_v.at[b, j0 + 1, 1]], add=True)
            return carry

        lax.fori_loop(0, G // 2, step, 0)
        if g + 2 < NG:
            pltpu.async_copy(sd_hbm.at[w, pl.ds((g + 2) * G, G)], idx_v.at[b],
                             semi)
    plsc.subcore_barrier()
    pltpu.sync_copy(acc.at[pl.ds(s * ROWS_PT, ROWS_PT)],
                    out_hbm.at[c, pl.ds(s * ROWS_PT, ROWS_PT)])


@functools.partial(
    pl.kernel,
    out_type=jax.ShapeDtypeStruct((NC, NPAD, DIM), jnp.float32),
    mesh=_mesh,
    scratch_types=[
        pltpu.VMEM_SHARED((NPAD, DIM), jnp.float32),
        pltpu.VMEM((CPW, CHUNK), jnp.int32),
        pltpu.VMEM((CHUNK, DIM), jnp.float32),
    ],
)
def _deg_kernel(dst_hbm, ones_hbm, zeros_hbm, out_hbm, acc, dst_v, ones_v):
    c = lax.axis_index("c")
    s = lax.axis_index("s")
    w = s * NC + c
    pltpu.sync_copy(zeros_hbm.at[pl.ds(s * ROWS_PT, ROWS_PT)],
                    acc.at[pl.ds(s * ROWS_PT, ROWS_PT)])
    pltpu.sync_copy(ones_hbm, ones_v)
    pltpu.sync_copy(dst_hbm.at[w], dst_v)
    plsc.subcore_barrier()

    def step(j, carry):
        pltpu.sync_copy(ones_v, acc.at[dst_v.at[j]], add=True)
        return carry

    lax.fori_loop(0, CPW, step, 0)
    plsc.subcore_barrier()
    pltpu.sync_copy(acc.at[pl.ds(s * ROWS_PT, ROWS_PT)],
                    out_hbm.at[c, pl.ds(s * ROWS_PT, ROWS_PT)])


def _dinv(degp_ref):
    deg = degp_ref[0] + degp_ref[1]
    return lax.rsqrt(deg[:, 0:1] + 1.0)


def _mm1_body(x_ref, w_ref, degp_ref, out_ref):
    out_ref[...] = (
        jnp.dot(x_ref[...], w_ref[...], preferred_element_type=jnp.float32)
        * _dinv(degp_ref)
    )


def _mm2_body(agg_ref, hs_ref, degp_ref, b_ref, w_ref, out_ref):
    di = _dinv(degp_ref)
    h = jnp.maximum((agg_ref[0] + agg_ref[1] + hs_ref[...]) * di + b_ref[...], 0.0)
    out_ref[...] = (
        jnp.dot(h, w_ref[...], preferred_element_type=jnp.float32) * di
    )


def _head_body(agg_ref, hs_ref, degp_ref, b2_ref, wf1_ref, bf1_ref,
               wf2_ref, bf2_ref, out_ref):
    di = _dinv(degp_ref)
    h = jnp.maximum((agg_ref[0] + agg_ref[1] + hs_ref[...]) * di + b2_ref[...], 0.0)
    h = jnp.maximum(
        jnp.dot(h, wf1_ref[...], preferred_element_type=jnp.float32) + bf1_ref[...],
        0.0,
    )
    out_ref[...] = (
        jnp.dot(h, wf2_ref[...], preferred_element_type=jnp.float32) + bf2_ref[...]
    )


def _mm1(x, W, degp):
    return pl.pallas_call(
        _mm1_body,
        grid=(NPAD // BT,),
        in_specs=[
            pl.BlockSpec((BT, DIM), lambda i: (i, 0)),
            pl.BlockSpec((DIM, DIM), lambda i: (0, 0)),
            pl.BlockSpec((NC, BT, DIM), lambda i: (0, i, 0)),
        ],
        out_specs=pl.BlockSpec((BT, DIM), lambda i: (i, 0)),
        out_shape=jax.ShapeDtypeStruct((NPAD, DIM), jnp.float32),
    )(x, W, degp)


def _mm2(agg, hs, degp, b, W):
    return pl.pallas_call(
        _mm2_body,
        grid=(NPAD // BT,),
        in_specs=[
            pl.BlockSpec((NC, BT, DIM), lambda i: (0, i, 0)),
            pl.BlockSpec((BT, DIM), lambda i: (i, 0)),
            pl.BlockSpec((NC, BT, DIM), lambda i: (0, i, 0)),
            pl.BlockSpec((1, DIM), lambda i: (0, 0)),
            pl.BlockSpec((DIM, DIM), lambda i: (0, 0)),
        ],
        out_specs=pl.BlockSpec((BT, DIM), lambda i: (i, 0)),
        out_shape=jax.ShapeDtypeStruct((NPAD, DIM), jnp.float32),
    )(agg, hs, degp, b, W)


def _head(agg, hs, degp, b2, Wf1, bf1, Wf2, bf2):
    return pl.pallas_call(
        _head_body,
        grid=(NPAD // BT,),
        in_specs=[
            pl.BlockSpec((NC, BT, DIM), lambda i: (0, i, 0)),
            pl.BlockSpec((BT, DIM), lambda i: (i, 0)),
            pl.BlockSpec((NC, BT, DIM), lambda i: (0, i, 0)),
            pl.BlockSpec((1, DIM), lambda i: (0, 0)),
            pl.BlockSpec((DIM, FC), lambda i: (0, 0)),
            pl.BlockSpec((1, FC), lambda i: (0, 0)),
            pl.BlockSpec((FC, 1), lambda i: (0, 0)),
            pl.BlockSpec((1, 1), lambda i: (0, 0)),
        ],
        out_specs=pl.BlockSpec((BT, 1), lambda i: (i, 0)),
        out_shape=jax.ShapeDtypeStruct((NPAD, 1), jnp.float32),
    )(agg, hs, degp, b2, Wf1, bf1, Wf2, bf2)


@jax.jit
def kernel(x, edge_index, W1, b1, W2, b2, Wf1, bf1, Wf2, bf2):
    src = edge_index[0].astype(jnp.int32)
    dst = edge_index[1].astype(jnp.int32)
    npad_edges = EPAD - N_EDGES
    pad_ids = jnp.arange(npad_edges, dtype=jnp.int32)
    src_p = jnp.concatenate([src, pad_ids % NPAD])
    dst_p = jnp.concatenate([dst, N_NODES + pad_ids % (NPAD - N_NODES)])
    src_w = src_p.reshape(NW, CPW, CHUNK)
    dst_w = dst_p.reshape(NW, CPW, CHUNK)

    x_pad = jnp.pad(x, ((0, NPAD - N_NODES), (0, 0)))
    zeros_dim = jnp.zeros((NPAD, DIM), jnp.float32)
    ones_chunk = jnp.ones((CHUNK, DIM), jnp.float32)

    sd_w = jnp.stack([src_w, dst_w], axis=2)

    degp = _deg_kernel(dst_w, ones_chunk, zeros_dim)
    hs1 = _mm1(x_pad, W1, degp)
    agg1 = _agg_kernel(sd_w, hs1, zeros_dim)
    hs2 = _mm2(agg1, hs1, degp, b1.reshape(1, DIM), W2)
    agg2 = _agg_kernel(sd_w, hs2, zeros_dim)
    out = _head(agg2, hs2, degp, b2.reshape(1, DIM), Wf1,
                bf1.reshape(1, FC), Wf2, bf2.reshape(1, 1))
    return out[:N_NODES, 0]
